# Initial kernel scaffold; baseline (speedup 1.0000x reference)
import jax
import jax.numpy as jnp
from jax.experimental import pallas as pl


def kernel(features, memory_bank, influence_weight, distance_weight):
    raise NotImplementedError("write your pallas kernel here")


# speedup vs baseline: 4.3816x; 4.3816x over previous
import functools

import jax
import jax.numpy as jnp
from jax import lax
from jax.experimental import pallas as pl
from jax.experimental.pallas import tpu as pltpu
from jax.experimental.pallas import tpu_sc as plsc

_K = 9
_KP = 16
_NOISE_LO, _NOISE_HI = 0.01, 0.5
_QB = 128
_NW = 32


def _dist_topk_body(flat_ref, mb_ref, mbt_ref, idx_ref, dsig_ref,
                    y2_scr, s_scr):
    n = mb_ref.shape[0]
    qb = flat_ref.shape[0]

    @pl.when(pl.program_id(0) == 0)
    def _():
        mbt = mbt_ref[...]
        y2_scr[...] = jnp.sum(mbt * mbt, axis=0, keepdims=True)

    fb = flat_ref[...]
    x2 = jnp.sum(fb * fb, axis=1, keepdims=True)
    xy = lax.dot_general(fb, mb_ref[...], (((1,), (1,)), ((), ())),
                         preferred_element_type=jnp.float32)
    s_scr[...] = (x2 + y2_scr[...]) - 2.0 * xy

    dsum = jnp.zeros((qb, 1), jnp.float32)
    idx_cols = []
    for _ in range(_K):
        w = s_scr[...]
        m = jnp.min(w, axis=1, keepdims=True)
        col = lax.broadcasted_iota(jnp.int32, (qb, n), 1)
        ij = jnp.min(jnp.where(w == m, col, n), axis=1, keepdims=True)
        idx_cols.append(ij)
        dsum = dsum + jnp.sqrt(jnp.maximum(m, 1e-12))
        s_scr[...] = jnp.where(col == ij, jnp.inf, w)

    idx_ref[...] = jnp.concatenate(idx_cols + [idx_cols[0]] * (_KP - _K),
                                   axis=1)
    dsig_ref[...] = dsum * (1.0 / _K)


def _dist_topk(flat, mb):
    q, c = flat.shape
    n = mb.shape[0]
    return pl.pallas_call(
        _dist_topk_body,
        grid=(q // _QB,),
        in_specs=[
            pl.BlockSpec((_QB, c), lambda i: (i, 0)),
            pl.BlockSpec((n, c), lambda i: (0, 0)),
            pl.BlockSpec((c, n), lambda i: (0, 0)),
        ],
        out_specs=[
            pl.BlockSpec((_QB, _KP), lambda i: (i, 0)),
            pl.BlockSpec((_QB, 1), lambda i: (i, 0)),
        ],
        out_shape=[
            jax.ShapeDtypeStruct((q, _KP), jnp.int32),
            jax.ShapeDtypeStruct((q, 1), jnp.float32),
        ],
        scratch_shapes=[
            pltpu.VMEM((1, n), jnp.float32),
            pltpu.VMEM((_QB, n), jnp.float32),
        ],
    )(flat, mb, mb.T)


def _gather_influence(flat, idx, mb):
    q, c = flat.shape
    qpt = q // _NW
    mesh = plsc.VectorSubcoreMesh(core_axis_name="c", subcore_axis_name="s")

    @functools.partial(
        pl.kernel,
        mesh=mesh,
        out_type=jax.ShapeDtypeStruct((q, c), jnp.float32),
        scratch_types=[
            pltpu.VMEM((qpt, c), jnp.float32),
            pltpu.VMEM((qpt, _KP), jnp.int32),
            pltpu.VMEM((_KP, c), jnp.float32),
            pltpu.VMEM((qpt, c), jnp.float32),
            pltpu.SemaphoreType.DMA,
        ],
    )
    def sc_kernel(flat_hbm, idx_hbm, mb_hbm, out_hbm,
                  flat_v, idx_v, rows_v, out_v, sem):
        wid = lax.axis_index("s") * 2 + lax.axis_index("c")
        q0 = wid * qpt
        pltpu.sync_copy(flat_hbm.at[pl.ds(q0, qpt)], flat_v)
        pltpu.sync_copy(idx_hbm.at[pl.ds(q0, qpt)], idx_v)

        def body(qi, carry):
            pltpu.async_copy(mb_hbm.at[idx_v.at[qi]], rows_v, sem).wait()
            for cc in range(c // 16):
                sl = pl.ds(cc * 16, 16)
                f = flat_v[qi, sl]
                a = jnp.abs(f - rows_v[0, sl])
                for j in range(1, _K):
                    a = a + jnp.abs(f - rows_v[j, sl])
                out_v[qi, sl] = a
            return carry

        lax.fori_loop(0, qpt, body, 0)
        pltpu.sync_copy(out_v, out_hbm.at[pl.ds(q0, qpt)])

    return sc_kernel(flat, idx, mb)


def _finalize_body(flat_ref, infl_ref, dsig_ref, w_ref, dw_ref, noise_ref,
                   out_ref, im_ref, ns_ref):
    q, c = flat_ref.shape
    infl = infl_ref[...] * (1.0 / _K) * w_ref[...]
    rm = jnp.mean(infl, axis=1, keepdims=True)
    cen = infl - rm
    std = jnp.sqrt(jnp.sum(cen * cen, axis=1, keepdims=True) / (c - 1))
    inorm = cen / (std + 1e-8)

    d = dsig_ref[...]
    dm = jnp.sum(d) * (1.0 / q)
    dc = d - dm
    dvar = jnp.sum(dc * dc) * (c / (q * c - 1.0))
    dnorm = dc / (jnp.sqrt(dvar) + 1e-8)

    comb = inorm + dw_ref[...] * dnorm
    nstd = _NOISE_LO + (_NOISE_HI - _NOISE_LO) * jax.nn.sigmoid(comb)
    out_ref[...] = flat_ref[...] + noise_ref[...] * nstd
    im_ref[...] = rm
    ns_ref[...] = jnp.mean(nstd, axis=1, keepdims=True)


def _finalize(flat, infl_sum, dsig, w, dw, noise):
    q, c = flat.shape
    return pl.pallas_call(
        _finalize_body,
        out_shape=[
            jax.ShapeDtypeStruct((q, c), jnp.float32),
            jax.ShapeDtypeStruct((q, 1), jnp.float32),
            jax.ShapeDtypeStruct((q, 1), jnp.float32),
        ],
    )(flat, infl_sum, dsig, w.reshape(1, c), dw.reshape(1, 1), noise)


def kernel(features, memory_bank, influence_weight, distance_weight):
    b, c, h, w = features.shape
    q = b * h * w
    flat = jnp.transpose(features, (0, 2, 3, 1)).reshape(q, c)

    idx, dsig = _dist_topk(flat, memory_bank)
    infl_sum = _gather_influence(flat, idx, memory_bank)

    noise = jax.random.normal(jax.random.key(1234), flat.shape,
                              dtype=flat.dtype)
    noised_flat, im, ns = _finalize(flat, infl_sum, dsig,
                                    influence_weight, distance_weight, noise)

    noised_features = jnp.transpose(noised_flat.reshape(b, h, w, c),
                                    (0, 3, 1, 2))
    return (noised_features, im.reshape(b, h, w), ns.reshape(b, h, w))


# speedup vs baseline: 5.4567x; 1.2454x over previous
import functools

import jax
import jax.numpy as jnp
from jax import lax
from jax.experimental import pallas as pl
from jax.experimental.pallas import tpu as pltpu
from jax.experimental.pallas import tpu_sc as plsc

_K = 9
_KP = 16
_NOISE_LO, _NOISE_HI = 0.01, 0.5
_QB = 128
_NW = 32


def _dist_topk_body(flat_ref, mb_ref, mbt_ref, idx_ref, dsig_ref,
                    y2_scr, s_scr):
    n = mb_ref.shape[0]
    qb = flat_ref.shape[0]

    @pl.when(pl.program_id(0) == 0)
    def _():
        mbt = mbt_ref[...]
        y2_scr[...] = jnp.sum(mbt * mbt, axis=0, keepdims=True)

    fb = flat_ref[...]
    x2 = jnp.sum(fb * fb, axis=1, keepdims=True)
    xy = lax.dot_general(fb, mb_ref[...], (((1,), (1,)), ((), ())),
                         preferred_element_type=jnp.float32)
    s_scr[...] = (x2 + y2_scr[...]) - 2.0 * xy

    dsum = jnp.zeros((qb, 1), jnp.float32)
    idx_cols = []
    for _ in range(_K):
        w = s_scr[...]
        m = jnp.min(w, axis=1, keepdims=True)
        col = lax.broadcasted_iota(jnp.int32, (qb, n), 1)
        ij = jnp.min(jnp.where(w == m, col, n), axis=1, keepdims=True)
        idx_cols.append(ij)
        dsum = dsum + jnp.sqrt(jnp.maximum(m, 1e-12))
        s_scr[...] = jnp.where(col == ij, jnp.inf, w)

    idx_ref[...] = jnp.concatenate(idx_cols, axis=1)
    dsig_ref[...] = dsum * (1.0 / _K)


def _dist_topk(flat, mb):
    q, c = flat.shape
    n = mb.shape[0]
    return pl.pallas_call(
        _dist_topk_body,
        grid=(q // _QB,),
        in_specs=[
            pl.BlockSpec((_QB, c), lambda i: (i, 0)),
            pl.BlockSpec((n, c), lambda i: (0, 0)),
            pl.BlockSpec((c, n), lambda i: (0, 0)),
        ],
        out_specs=[
            pl.BlockSpec((_QB, _K), lambda i: (i, 0)),
            pl.BlockSpec((_QB, 1), lambda i: (i, 0)),
        ],
        out_shape=[
            jax.ShapeDtypeStruct((q, _K), jnp.int32),
            jax.ShapeDtypeStruct((q, 1), jnp.float32),
        ],
        scratch_shapes=[
            pltpu.VMEM((1, n), jnp.float32),
            pltpu.VMEM((_QB, n), jnp.float32),
        ],
    )(flat, mb, mb.T)


_GQ = 8


def _gather_influence(flat, idx_flat, mb):
    q, c = flat.shape
    qpt = q // _NW
    ngrp = qpt // _GQ
    rows = _GQ * _K
    mesh = plsc.VectorSubcoreMesh(core_axis_name="c", subcore_axis_name="s")

    @functools.partial(
        pl.kernel,
        mesh=mesh,
        out_type=jax.ShapeDtypeStruct((q, c), jnp.float32),
        scratch_types=[
            pltpu.VMEM((qpt, c), jnp.float32),
            pltpu.VMEM((qpt * _K,), jnp.int32),
            pltpu.VMEM((rows, c), jnp.float32),
            pltpu.VMEM((rows, c), jnp.float32),
            pltpu.VMEM((qpt, c), jnp.float32),
            pltpu.SemaphoreType.DMA,
            pltpu.SemaphoreType.DMA,
        ],
    )
    def sc_kernel(flat_hbm, idx_hbm, mb_hbm, out_hbm,
                  flat_v, idx_v, rows0, rows1, out_v, sem0, sem1):
        wid = lax.axis_index("s") * 2 + lax.axis_index("c")
        q0 = wid * qpt
        pltpu.sync_copy(flat_hbm.at[pl.ds(q0, qpt)], flat_v)
        pltpu.sync_copy(idx_hbm.at[pl.ds(q0 * _K, qpt * _K)], idx_v)

        bufs = [rows0, rows1]
        sems = [sem0, sem1]

        def issue(g, b):
            return pltpu.async_copy(
                mb_hbm.at[idx_v.at[pl.ds(g * rows, rows)]], bufs[b], sems[b])

        def compute_group(g, buf):
            def qbody(qi, carry):
                qq = g * _GQ + qi
                for cc in range(c // 16):
                    sl = pl.ds(cc * 16, 16)
                    f = flat_v[qq, sl]
                    a = jnp.abs(f - buf[qi * _K, sl])
                    for j in range(1, _K):
                        a = a + jnp.abs(f - buf[qi * _K + j, sl])
                    out_v[qq, sl] = a
                return carry
            lax.fori_loop(0, _GQ, qbody, 0)

        cps = [issue(0, 0), None]
        for g in range(ngrp):
            b = g % 2
            cps[b].wait()
            if g + 1 < ngrp:
                cps[1 - b] = issue(g + 1, 1 - b)
            compute_group(g, bufs[b])

        pltpu.sync_copy(out_v, out_hbm.at[pl.ds(q0, qpt)])

    return sc_kernel(flat, idx_flat, mb)


def _finalize_body(flat_ref, infl_ref, dsig_ref, w_ref, dw_ref, noise_ref,
                   out_ref, im_ref, ns_ref):
    q, c = flat_ref.shape
    infl = infl_ref[...] * (1.0 / _K) * w_ref[...]
    rm = jnp.mean(infl, axis=1, keepdims=True)
    cen = infl - rm
    std = jnp.sqrt(jnp.sum(cen * cen, axis=1, keepdims=True) / (c - 1))
    inorm = cen / (std + 1e-8)

    d = dsig_ref[...]
    dm = jnp.sum(d) * (1.0 / q)
    dc = d - dm
    dvar = jnp.sum(dc * dc) * (c / (q * c - 1.0))
    dnorm = dc / (jnp.sqrt(dvar) + 1e-8)

    comb = inorm + dw_ref[...] * dnorm
    nstd = _NOISE_LO + (_NOISE_HI - _NOISE_LO) * jax.nn.sigmoid(comb)
    out_ref[...] = flat_ref[...] + noise_ref[...] * nstd
    im_ref[...] = rm
    ns_ref[...] = jnp.mean(nstd, axis=1, keepdims=True)


def _finalize(flat, infl_sum, dsig, w, dw, noise):
    q, c = flat.shape
    return pl.pallas_call(
        _finalize_body,
        out_shape=[
            jax.ShapeDtypeStruct((q, c), jnp.float32),
            jax.ShapeDtypeStruct((q, 1), jnp.float32),
            jax.ShapeDtypeStruct((q, 1), jnp.float32),
        ],
    )(flat, infl_sum, dsig, w.reshape(1, c), dw.reshape(1, 1), noise)


def kernel(features, memory_bank, influence_weight, distance_weight):
    b, c, h, w = features.shape
    q = b * h * w
    flat = jnp.transpose(features, (0, 2, 3, 1)).reshape(q, c)

    idx, dsig = _dist_topk(flat, memory_bank)
    infl_sum = _gather_influence(flat, idx.reshape(-1), memory_bank)

    noise = jax.random.normal(jax.random.key(1234), flat.shape,
                              dtype=flat.dtype)
    noised_flat, im, ns = _finalize(flat, infl_sum, dsig,
                                    influence_weight, distance_weight, noise)

    noised_features = jnp.transpose(noised_flat.reshape(b, h, w, c),
                                    (0, 3, 1, 2))
    return (noised_features, im.reshape(b, h, w), ns.reshape(b, h, w))


# speedup vs baseline: 5.6537x; 1.0361x over previous
import functools

import jax
import jax.numpy as jnp
from jax import lax
from jax.experimental import pallas as pl
from jax.experimental.pallas import tpu as pltpu
from jax.experimental.pallas import tpu_sc as plsc

_K = 9
_KP = 16
_NOISE_LO, _NOISE_HI = 0.01, 0.5
_QB = 128
_NW = 32


def _dist_topk_body(flat_ref, mb_ref, idx_ref, dsig_ref, y2_scr, s_scr):
    n = mb_ref.shape[0]
    qb = flat_ref.shape[0]
    c = mb_ref.shape[1]

    @pl.when(pl.program_id(0) == 0)
    def _():
        mbsq = mb_ref[...] * mb_ref[...]
        y2_scr[...] = lax.dot_general(
            jnp.ones((1, c), jnp.float32), mbsq, (((1,), (1,)), ((), ())),
            preferred_element_type=jnp.float32,
            precision=lax.Precision.HIGHEST)

    fb = flat_ref[...]
    x2 = jnp.sum(fb * fb, axis=1, keepdims=True)
    xy = lax.dot_general(fb, mb_ref[...], (((1,), (1,)), ((), ())),
                         preferred_element_type=jnp.float32)
    s_scr[...] = (x2 + y2_scr[...]) - 2.0 * xy

    dsum = jnp.zeros((qb, 1), jnp.float32)
    idx_cols = []
    for _ in range(_K):
        w = s_scr[...]
        m = jnp.min(w, axis=1, keepdims=True)
        col = lax.broadcasted_iota(jnp.int32, (qb, n), 1)
        ij = jnp.min(jnp.where(w == m, col, n), axis=1, keepdims=True)
        idx_cols.append(ij)
        dsum = dsum + jnp.sqrt(jnp.maximum(m, 1e-12))
        s_scr[...] = jnp.where(col == ij, jnp.inf, w)

    idx_ref[...] = jnp.concatenate(idx_cols, axis=1)
    dsig_ref[...] = dsum * (1.0 / _K)


def _dist_topk(flat, mb):
    q, c = flat.shape
    n = mb.shape[0]
    return pl.pallas_call(
        _dist_topk_body,
        grid=(q // _QB,),
        in_specs=[
            pl.BlockSpec((_QB, c), lambda i: (i, 0)),
            pl.BlockSpec((n, c), lambda i: (0, 0)),
        ],
        out_specs=[
            pl.BlockSpec((_QB, _K), lambda i: (i, 0)),
            pl.BlockSpec((_QB, 1), lambda i: (i, 0)),
        ],
        out_shape=[
            jax.ShapeDtypeStruct((q, _K), jnp.int32),
            jax.ShapeDtypeStruct((q, 1), jnp.float32),
        ],
        scratch_shapes=[
            pltpu.VMEM((1, n), jnp.float32),
            pltpu.VMEM((_QB, n), jnp.float32),
        ],
    )(flat, mb)


_GQ = 8


def _gather_influence(flat, idx_flat, mb):
    q, c = flat.shape
    qpt = q // _NW
    ngrp = qpt // _GQ
    rows = _GQ * _K
    mesh = plsc.VectorSubcoreMesh(core_axis_name="c", subcore_axis_name="s")

    @functools.partial(
        pl.kernel,
        mesh=mesh,
        out_type=jax.ShapeDtypeStruct((q, c), jnp.float32),
        scratch_types=[
            pltpu.VMEM((qpt, c), jnp.float32),
            pltpu.VMEM((qpt * _K,), jnp.int32),
            pltpu.VMEM((rows, c), jnp.float32),
            pltpu.VMEM((rows, c), jnp.float32),
            pltpu.VMEM((qpt, c), jnp.float32),
            pltpu.SemaphoreType.DMA,
            pltpu.SemaphoreType.DMA,
        ],
    )
    def sc_kernel(flat_hbm, idx_hbm, mb_hbm, out_hbm,
                  flat_v, idx_v, rows0, rows1, out_v, sem0, sem1):
        wid = lax.axis_index("s") * 2 + lax.axis_index("c")
        q0 = wid * qpt
        pltpu.sync_copy(flat_hbm.at[pl.ds(q0, qpt)], flat_v)
        pltpu.sync_copy(idx_hbm.at[pl.ds(q0 * _K, qpt * _K)], idx_v)

        bufs = [rows0, rows1]
        sems = [sem0, sem1]

        def issue(g, b):
            return pltpu.async_copy(
                mb_hbm.at[idx_v.at[pl.ds(g * rows, rows)]], bufs[b], sems[b])

        def compute_group(g, buf):
            def qbody(qi, carry):
                qq = g * _GQ + qi
                for cc in range(c // 16):
                    sl = pl.ds(cc * 16, 16)
                    f = flat_v[qq, sl]
                    a = jnp.abs(f - buf[qi * _K, sl])
                    for j in range(1, _K):
                        a = a + jnp.abs(f - buf[qi * _K + j, sl])
                    out_v[qq, sl] = a
                return carry
            lax.fori_loop(0, _GQ, qbody, 0)

        cps = [issue(0, 0), None]
        for g in range(ngrp):
            b = g % 2
            cps[b].wait()
            if g + 1 < ngrp:
                cps[1 - b] = issue(g + 1, 1 - b)
            compute_group(g, bufs[b])

        pltpu.sync_copy(out_v, out_hbm.at[pl.ds(q0, qpt)])

    return sc_kernel(flat, idx_flat, mb)


def _finalize_body(flat_ref, infl_ref, dsig_ref, w_ref, dw_ref, noise_ref,
                   out_ref, im_ref, ns_ref):
    q, c = flat_ref.shape
    infl = infl_ref[...] * (1.0 / _K) * w_ref[...]
    rm = jnp.mean(infl, axis=1, keepdims=True)
    cen = infl - rm
    std = jnp.sqrt(jnp.sum(cen * cen, axis=1, keepdims=True) / (c - 1))
    inorm = cen / (std + 1e-8)

    d = dsig_ref[...]
    dm = jnp.sum(d) * (1.0 / q)
    dc = d - dm
    dvar = jnp.sum(dc * dc) * (c / (q * c - 1.0))
    dnorm = dc / (jnp.sqrt(dvar) + 1e-8)

    comb = inorm + dw_ref[...] * dnorm
    nstd = _NOISE_LO + (_NOISE_HI - _NOISE_LO) * jax.nn.sigmoid(comb)
    out_ref[...] = flat_ref[...] + noise_ref[...] * nstd
    im_ref[...] = rm
    ns_ref[...] = jnp.mean(nstd, axis=1, keepdims=True)


def _finalize(flat, infl_sum, dsig, w, dw, noise):
    q, c = flat.shape
    return pl.pallas_call(
        _finalize_body,
        out_shape=[
            jax.ShapeDtypeStruct((q, c), jnp.float32),
            jax.ShapeDtypeStruct((q, 1), jnp.float32),
            jax.ShapeDtypeStruct((q, 1), jnp.float32),
        ],
    )(flat, infl_sum, dsig, w.reshape(1, c), dw.reshape(1, 1), noise)


def kernel(features, memory_bank, influence_weight, distance_weight):
    b, c, h, w = features.shape
    q = b * h * w
    flat = jnp.transpose(features, (0, 2, 3, 1)).reshape(q, c)

    idx, dsig = _dist_topk(flat, memory_bank)
    infl_sum = _gather_influence(flat, idx.reshape(-1), memory_bank)

    noise = jax.random.normal(jax.random.key(1234), flat.shape,
                              dtype=flat.dtype)
    noised_flat, im, ns = _finalize(flat, infl_sum, dsig,
                                    influence_weight, distance_weight, noise)

    noised_features = jnp.transpose(noised_flat.reshape(b, h, w, c),
                                    (0, 3, 1, 2))
    return (noised_features, im.reshape(b, h, w), ns.reshape(b, h, w))


# speedup vs baseline: 6.4030x; 1.1325x over previous
import functools

import jax
import jax.numpy as jnp
from jax import lax
from jax.experimental import pallas as pl
from jax.experimental.pallas import tpu as pltpu
from jax.experimental.pallas import tpu_sc as plsc

_K = 9
_KP = 16
_NOISE_LO, _NOISE_HI = 0.01, 0.5
_QB = 128
_NW = 32


def _dist_topk_body(flat_ref, mb_ref, idx_ref, dsig_ref, y2_scr, s_scr,
                    col_scr):
    n = mb_ref.shape[0]
    qb = flat_ref.shape[0]
    c = mb_ref.shape[1]

    @pl.when(pl.program_id(0) == 0)
    def _():
        mbsq = mb_ref[...] * mb_ref[...]
        y2_scr[...] = lax.dot_general(
            jnp.ones((1, c), jnp.float32), mbsq, (((1,), (1,)), ((), ())),
            preferred_element_type=jnp.float32,
            precision=lax.Precision.HIGHEST)

    fb = flat_ref[...]
    x2 = jnp.sum(fb * fb, axis=1, keepdims=True)
    xy = lax.dot_general(fb, mb_ref[...], (((1,), (1,)), ((), ())),
                         preferred_element_type=jnp.float32)
    s_scr[...] = (x2 + y2_scr[...]) - 2.0 * xy

    col_scr[...] = lax.broadcasted_iota(
        jnp.int32, (qb, n), 1).astype(jnp.float32)

    dsum = jnp.zeros((qb, 1), jnp.float32)
    idx_cols = []
    for _ in range(_K):
        w = s_scr[...]
        m = jnp.min(w, axis=1, keepdims=True)
        col = col_scr[...]
        ij = jnp.min(jnp.where(w == m, col, float(n)), axis=1,
                     keepdims=True)
        idx_cols.append(ij.astype(jnp.int32))
        dsum = dsum + jnp.sqrt(jnp.maximum(m, 1e-12))
        s_scr[...] = jnp.where(col == ij, jnp.inf, w)

    idx_ref[...] = jnp.concatenate(idx_cols, axis=1)
    dsig_ref[...] = dsum * (1.0 / _K)


def _dist_topk(flat, mb):
    q, c = flat.shape
    n = mb.shape[0]
    return pl.pallas_call(
        _dist_topk_body,
        grid=(q // _QB,),
        in_specs=[
            pl.BlockSpec((_QB, c), lambda i: (i, 0)),
            pl.BlockSpec((n, c), lambda i: (0, 0)),
        ],
        out_specs=[
            pl.BlockSpec((_QB, _K), lambda i: (i, 0)),
            pl.BlockSpec((_QB, 1), lambda i: (i, 0)),
        ],
        out_shape=[
            jax.ShapeDtypeStruct((q, _K), jnp.int32),
            jax.ShapeDtypeStruct((q, 1), jnp.float32),
        ],
        scratch_shapes=[
            pltpu.VMEM((1, n), jnp.float32),
            pltpu.VMEM((_QB, n), jnp.float32),
            pltpu.VMEM((_QB, n), jnp.float32),
        ],
    )(flat, mb)


_GQ = 8


def _gather_influence(flat, idx_flat, mb):
    q, c = flat.shape
    qpt = q // _NW
    ngrp = qpt // _GQ
    rows = _GQ * _K
    mesh = plsc.VectorSubcoreMesh(core_axis_name="c", subcore_axis_name="s")

    @functools.partial(
        pl.kernel,
        mesh=mesh,
        out_type=jax.ShapeDtypeStruct((q, c), jnp.float32),
        scratch_types=[
            pltpu.VMEM((qpt, c), jnp.float32),
            pltpu.VMEM((qpt * _K,), jnp.int32),
            pltpu.VMEM((rows, c), jnp.float32),
            pltpu.VMEM((rows, c), jnp.float32),
            pltpu.VMEM((qpt, c), jnp.float32),
            pltpu.SemaphoreType.DMA,
            pltpu.SemaphoreType.DMA,
        ],
    )
    def sc_kernel(flat_hbm, idx_hbm, mb_hbm, out_hbm,
                  flat_v, idx_v, rows0, rows1, out_v, sem0, sem1):
        wid = lax.axis_index("s") * 2 + lax.axis_index("c")
        q0 = wid * qpt
        pltpu.sync_copy(flat_hbm.at[pl.ds(q0, qpt)], flat_v)
        pltpu.sync_copy(idx_hbm.at[pl.ds(q0 * _K, qpt * _K)], idx_v)

        bufs = [rows0, rows1]
        sems = [sem0, sem1]

        def issue(g, b):
            return pltpu.async_copy(
                mb_hbm.at[idx_v.at[pl.ds(g * rows, rows)]], bufs[b], sems[b])

        def compute_group(g, buf):
            def qbody(qi, carry):
                qq = g * _GQ + qi
                for cc in range(c // 16):
                    sl = pl.ds(cc * 16, 16)
                    f = flat_v[qq, sl]
                    a = jnp.abs(f - buf[qi * _K, sl])
                    for j in range(1, _K):
                        a = a + jnp.abs(f - buf[qi * _K + j, sl])
                    out_v[qq, sl] = a
                return carry
            lax.fori_loop(0, _GQ, qbody, 0)

        cps = [issue(0, 0), None]
        for g in range(ngrp):
            b = g % 2
            cps[b].wait()
            if g + 1 < ngrp:
                cps[1 - b] = issue(g + 1, 1 - b)
            compute_group(g, bufs[b])

        pltpu.sync_copy(out_v, out_hbm.at[pl.ds(q0, qpt)])

    return sc_kernel(flat, idx_flat, mb)


def _finalize_body(flat_ref, infl_ref, dsig_ref, w_ref, dw_ref, noise_ref,
                   out_ref, im_ref, ns_ref):
    q, c = flat_ref.shape
    infl = infl_ref[...] * (1.0 / _K) * w_ref[...]
    rm = jnp.mean(infl, axis=1, keepdims=True)
    cen = infl - rm
    std = jnp.sqrt(jnp.sum(cen * cen, axis=1, keepdims=True) / (c - 1))
    inorm = cen / (std + 1e-8)

    d = dsig_ref[...]
    dm = jnp.sum(d) * (1.0 / q)
    dc = d - dm
    dvar = jnp.sum(dc * dc) * (c / (q * c - 1.0))
    dnorm = dc / (jnp.sqrt(dvar) + 1e-8)

    comb = inorm + dw_ref[...] * dnorm
    nstd = _NOISE_LO + (_NOISE_HI - _NOISE_LO) * jax.nn.sigmoid(comb)
    out_ref[...] = flat_ref[...] + noise_ref[...] * nstd
    im_ref[...] = rm
    ns_ref[...] = jnp.mean(nstd, axis=1, keepdims=True)


def _finalize(flat, infl_sum, dsig, w, dw, noise):
    q, c = flat.shape
    return pl.pallas_call(
        _finalize_body,
        out_shape=[
            jax.ShapeDtypeStruct((q, c), jnp.float32),
            jax.ShapeDtypeStruct((q, 1), jnp.float32),
            jax.ShapeDtypeStruct((q, 1), jnp.float32),
        ],
    )(flat, infl_sum, dsig, w.reshape(1, c), dw.reshape(1, 1), noise)


def kernel(features, memory_bank, influence_weight, distance_weight):
    b, c, h, w = features.shape
    q = b * h * w
    flat = jnp.transpose(features, (0, 2, 3, 1)).reshape(q, c)

    idx, dsig = _dist_topk(flat, memory_bank)
    infl_sum = _gather_influence(flat, idx.reshape(-1), memory_bank)

    noise = jax.random.normal(jax.random.key(1234), flat.shape,
                              dtype=flat.dtype)
    noised_flat, im, ns = _finalize(flat, infl_sum, dsig,
                                    influence_weight, distance_weight, noise)

    noised_features = jnp.transpose(noised_flat.reshape(b, h, w, c),
                                    (0, 3, 1, 2))
    return (noised_features, im.reshape(b, h, w), ns.reshape(b, h, w))


# speedup vs baseline: 7.2023x; 1.1248x over previous
import functools

import jax
import jax.numpy as jnp
from jax import lax
from jax.experimental import pallas as pl
from jax.experimental.pallas import tpu as pltpu
from jax.experimental.pallas import tpu_sc as plsc

_K = 9
_KP = 16
_NOISE_LO, _NOISE_HI = 0.01, 0.5
_QB = 256
_NW = 32


def _dist_topk_body(flat_ref, mb_ref, idx_ref, dsig_ref, y2_scr, s_scr,
                    col_scr):
    n = mb_ref.shape[0]
    qb = flat_ref.shape[0]
    c = mb_ref.shape[1]

    @pl.when(pl.program_id(0) == 0)
    def _():
        mbsq = mb_ref[...] * mb_ref[...]
        y2_scr[...] = lax.dot_general(
            jnp.ones((1, c), jnp.float32), mbsq, (((1,), (1,)), ((), ())),
            preferred_element_type=jnp.float32,
            precision=lax.Precision.HIGHEST)

    fb = flat_ref[...]
    x2 = jnp.sum(fb * fb, axis=1, keepdims=True)
    xy = lax.dot_general(fb, mb_ref[...], (((1,), (1,)), ((), ())),
                         preferred_element_type=jnp.float32)
    s_scr[...] = (x2 + y2_scr[...]) - 2.0 * xy

    col_scr[...] = lax.broadcasted_iota(
        jnp.int32, (qb, n), 1).astype(jnp.float32)

    dsum = jnp.zeros((qb, 1), jnp.float32)
    idx_cols = []
    for _ in range(_K):
        w = s_scr[...]
        m = jnp.min(w, axis=1, keepdims=True)
        col = col_scr[...]
        ij = jnp.min(jnp.where(w == m, col, float(n)), axis=1,
                     keepdims=True)
        idx_cols.append(ij.astype(jnp.int32))
        dsum = dsum + jnp.sqrt(jnp.maximum(m, 1e-12))
        s_scr[...] = jnp.where(col == ij, jnp.inf, w)

    idx_ref[...] = jnp.concatenate(idx_cols, axis=1)
    dsig_ref[...] = dsum * (1.0 / _K)


def _dist_topk(flat, mb):
    q, c = flat.shape
    n = mb.shape[0]
    return pl.pallas_call(
        _dist_topk_body,
        grid=(q // _QB,),
        in_specs=[
            pl.BlockSpec((_QB, c), lambda i: (i, 0)),
            pl.BlockSpec((n, c), lambda i: (0, 0)),
        ],
        out_specs=[
            pl.BlockSpec((_QB, _K), lambda i: (i, 0)),
            pl.BlockSpec((_QB, 1), lambda i: (i, 0)),
        ],
        out_shape=[
            jax.ShapeDtypeStruct((q, _K), jnp.int32),
            jax.ShapeDtypeStruct((q, 1), jnp.float32),
        ],
        scratch_shapes=[
            pltpu.VMEM((1, n), jnp.float32),
            pltpu.VMEM((_QB, n), jnp.float32),
            pltpu.VMEM((_QB, n), jnp.float32),
        ],
    )(flat, mb)


_GQ = 8


def _gather_influence(flat, idx_flat, mb):
    q, c = flat.shape
    qpt = q // _NW
    ngrp = qpt // _GQ
    rows = _GQ * _K
    mesh = plsc.VectorSubcoreMesh(core_axis_name="c", subcore_axis_name="s")

    @functools.partial(
        pl.kernel,
        mesh=mesh,
        out_type=jax.ShapeDtypeStruct((q, c), jnp.float32),
        scratch_types=[
            pltpu.VMEM((qpt, c), jnp.float32),
            pltpu.VMEM((qpt * _K,), jnp.int32),
            pltpu.VMEM((rows, c), jnp.float32),
            pltpu.VMEM((rows, c), jnp.float32),
            pltpu.VMEM((qpt, c), jnp.float32),
            pltpu.SemaphoreType.DMA,
            pltpu.SemaphoreType.DMA,
        ],
    )
    def sc_kernel(flat_hbm, idx_hbm, mb_hbm, out_hbm,
                  flat_v, idx_v, rows0, rows1, out_v, sem0, sem1):
        wid = lax.axis_index("s") * 2 + lax.axis_index("c")
        q0 = wid * qpt
        pltpu.sync_copy(flat_hbm.at[pl.ds(q0, qpt)], flat_v)
        pltpu.sync_copy(idx_hbm.at[pl.ds(q0 * _K, qpt * _K)], idx_v)

        bufs = [rows0, rows1]
        sems = [sem0, sem1]

        def issue(g, b):
            return pltpu.async_copy(
                mb_hbm.at[idx_v.at[pl.ds(g * rows, rows)]], bufs[b], sems[b])

        def compute_group(g, buf):
            def qbody(qi, carry):
                qq = g * _GQ + qi
                for cc in range(c // 16):
                    sl = pl.ds(cc * 16, 16)
                    f = flat_v[qq, sl]
                    a = jnp.abs(f - buf[qi * _K, sl])
                    for j in range(1, _K):
                        a = a + jnp.abs(f - buf[qi * _K + j, sl])
                    out_v[qq, sl] = a
                return carry
            lax.fori_loop(0, _GQ, qbody, 0)

        cps = [issue(0, 0), None]
        for g in range(ngrp):
            b = g % 2
            cps[b].wait()
            if g + 1 < ngrp:
                cps[1 - b] = issue(g + 1, 1 - b)
            compute_group(g, bufs[b])

        pltpu.sync_copy(out_v, out_hbm.at[pl.ds(q0, qpt)])

    return sc_kernel(flat, idx_flat, mb)


def _finalize_body(flat_ref, infl_ref, dsig_ref, w_ref, dw_ref, noise_ref,
                   out_ref, im_ref, ns_ref):
    q, c = flat_ref.shape
    infl = infl_ref[...] * (1.0 / _K) * w_ref[...]
    rm = jnp.mean(infl, axis=1, keepdims=True)
    cen = infl - rm
    std = jnp.sqrt(jnp.sum(cen * cen, axis=1, keepdims=True) / (c - 1))
    inorm = cen / (std + 1e-8)

    d = dsig_ref[...]
    dm = jnp.sum(d) * (1.0 / q)
    dc = d - dm
    dvar = jnp.sum(dc * dc) * (c / (q * c - 1.0))
    dnorm = dc / (jnp.sqrt(dvar) + 1e-8)

    comb = inorm + dw_ref[...] * dnorm
    nstd = _NOISE_LO + (_NOISE_HI - _NOISE_LO) * jax.nn.sigmoid(comb)
    out_ref[...] = flat_ref[...] + noise_ref[...] * nstd
    im_ref[...] = rm
    ns_ref[...] = jnp.mean(nstd, axis=1, keepdims=True)


def _finalize(flat, infl_sum, dsig, w, dw, noise):
    q, c = flat.shape
    return pl.pallas_call(
        _finalize_body,
        out_shape=[
            jax.ShapeDtypeStruct((q, c), jnp.float32),
            jax.ShapeDtypeStruct((q, 1), jnp.float32),
            jax.ShapeDtypeStruct((q, 1), jnp.float32),
        ],
    )(flat, infl_sum, dsig, w.reshape(1, c), dw.reshape(1, 1), noise)


def kernel(features, memory_bank, influence_weight, distance_weight):
    b, c, h, w = features.shape
    q = b * h * w
    flat = jnp.transpose(features, (0, 2, 3, 1)).reshape(q, c)

    idx, dsig = _dist_topk(flat, memory_bank)
    infl_sum = _gather_influence(flat, idx.reshape(-1), memory_bank)

    noise = jax.random.normal(jax.random.key(1234), flat.shape,
                              dtype=flat.dtype)
    noised_flat, im, ns = _finalize(flat, infl_sum, dsig,
                                    influence_weight, distance_weight, noise)

    noised_features = jnp.transpose(noised_flat.reshape(b, h, w, c),
                                    (0, 3, 1, 2))
    return (noised_features, im.reshape(b, h, w), ns.reshape(b, h, w))
